# baseline (device time: 105171 ns/iter reference)
import jax
import jax.numpy as jnp
from jax import lax
from jax.experimental import pallas as pl
from jax.experimental.pallas import tpu as pltpu

LAG = 2
RING = 4


def kernel(Q, K, V):
    b, s, h, d = Q.shape
    scale = d ** -0.5

    def body(q_hbm, k_hbm, v_hbm, o_hbm,
             qs, ks, vs, ostage, qT, kT, vT, krem, vrem, uring, lring,
             in_sems, o_sems, ksend, krecv, vsend, vrecv):
        i = pl.program_id(0)
        my_x = lax.axis_index("x")
        my_y = lax.axis_index("y")
        my_z = lax.axis_index("z")
        partner = (1 - my_x, my_y, my_z)

        def in_dma(j, slot):
            return [
                pltpu.make_async_copy(
                    ref.at[0, :, j, :], stage.at[slot], in_sems.at[slot, t])
                for t, (ref, stage) in enumerate(
                    [(q_hbm, qs), (k_hbm, ks), (v_hbm, vs)])
            ]

        @pl.when(i == 0)
        def _():
            barrier_sem = pltpu.get_barrier_semaphore()
            pl.semaphore_signal(
                barrier_sem, inc=1, device_id=partner,
                device_id_type=pl.DeviceIdType.MESH,
            )
            pl.semaphore_wait(barrier_sem, 1)
            for c in in_dma(0, 0):
                c.start()

        @pl.when(i < h)
        def _():
            slot = lax.rem(i, 2)
            for c in in_dma(i, slot):
                c.wait()

            @pl.when(i + 1 < h)
            def _():
                for c in in_dma(i + 1, lax.rem(i + 1, 2)):
                    c.start()

            qT[i] = (qs[slot] * scale).astype(jnp.bfloat16)
            kT[i] = ks[slot].astype(jnp.bfloat16)
            vT[i] = vs[slot].astype(jnp.bfloat16)
            rdma_k = pltpu.make_async_remote_copy(
                src_ref=kT.at[i], dst_ref=krem.at[i],
                send_sem=ksend.at[i], recv_sem=krecv.at[i],
                device_id=partner, device_id_type=pl.DeviceIdType.MESH,
            )
            rdma_v = pltpu.make_async_remote_copy(
                src_ref=vT.at[i], dst_ref=vrem.at[i],
                send_sem=vsend.at[i], recv_sem=vrecv.at[i],
                device_id=partner, device_id_type=pl.DeviceIdType.MESH,
            )
            rdma_k.start()
            rdma_v.start()

            r = lax.rem(i, RING)
            s1 = lax.dot_general(
                qT[i], kT[i], (((1,), (1,)), ((), ())),
                preferred_element_type=jnp.float32,
            )
            p1 = jnp.exp(s1).astype(jnp.bfloat16)
            ones = jnp.ones((s, 128), jnp.bfloat16)
            uring[r] = lax.dot_general(
                p1, vT[i], (((1,), (0,)), ((), ())),
                preferred_element_type=jnp.float32,
            )
            lring[r] = lax.dot_general(
                p1, ones, (((1,), (0,)), ((), ())),
                preferred_element_type=jnp.float32,
            )

        @pl.when(i >= LAG)
        def _():
            m = i - LAG
            sm = lax.rem(m, 2)
            rm = lax.rem(m, RING)
            rdma_k = pltpu.make_async_remote_copy(
                src_ref=kT.at[m], dst_ref=krem.at[m],
                send_sem=ksend.at[m], recv_sem=krecv.at[m],
                device_id=partner, device_id_type=pl.DeviceIdType.MESH,
            )
            rdma_v = pltpu.make_async_remote_copy(
                src_ref=vT.at[m], dst_ref=vrem.at[m],
                send_sem=vsend.at[m], recv_sem=vrecv.at[m],
                device_id=partner, device_id_type=pl.DeviceIdType.MESH,
            )
            rdma_k.wait_recv()
            rdma_v.wait_recv()
            s2 = lax.dot_general(
                qT[m], krem[m], (((1,), (1,)), ((), ())),
                preferred_element_type=jnp.float32,
            )
            p2 = jnp.exp(s2).astype(jnp.bfloat16)
            ones = jnp.ones((s, 128), jnp.bfloat16)
            u2 = lax.dot_general(
                p2, vrem[m], (((1,), (0,)), ((), ())),
                preferred_element_type=jnp.float32,
            )
            l2 = lax.dot_general(
                p2, ones, (((1,), (0,)), ((), ())),
                preferred_element_type=jnp.float32,
            )

            def o_dma(j, slot):
                return pltpu.make_async_copy(
                    ostage.at[slot], o_hbm.at[0, :, j, :], o_sems.at[slot])

            @pl.when(m >= 2)
            def _():
                o_dma(m - 2, sm).wait()

            ostage[sm] = (uring[rm] + u2) / (lring[rm][:, 0:1] + l2[:, 0:1])
            o_dma(m, sm).start()

            rdma_k.wait_send()
            rdma_v.wait_send()

            @pl.when(m == h - 1)
            def _():
                o_dma(m - 1, lax.rem(m - 1, 2)).wait()
                o_dma(m, sm).wait()

    out = pl.pallas_call(
        body,
        grid=(h + LAG,),
        out_shape=jax.ShapeDtypeStruct((b, s, h, d), jnp.float32),
        in_specs=[pl.BlockSpec(memory_space=pl.ANY)] * 3,
        out_specs=pl.BlockSpec(memory_space=pl.ANY),
        scratch_shapes=[
            pltpu.VMEM((2, s, d), jnp.float32),
            pltpu.VMEM((2, s, d), jnp.float32),
            pltpu.VMEM((2, s, d), jnp.float32),
            pltpu.VMEM((2, s, d), jnp.float32),
            pltpu.VMEM((h, s, d), jnp.bfloat16),
            pltpu.VMEM((h, s, d), jnp.bfloat16),
            pltpu.VMEM((h, s, d), jnp.bfloat16),
            pltpu.VMEM((h, s, d), jnp.bfloat16),
            pltpu.VMEM((h, s, d), jnp.bfloat16),
            pltpu.VMEM((RING, s, d), jnp.float32),
            pltpu.VMEM((RING, s, 128), jnp.float32),
            pltpu.SemaphoreType.DMA((2, 3)),
            pltpu.SemaphoreType.DMA((2,)),
            pltpu.SemaphoreType.DMA((h,)),
            pltpu.SemaphoreType.DMA((h,)),
            pltpu.SemaphoreType.DMA((h,)),
            pltpu.SemaphoreType.DMA((h,)),
        ],
        compiler_params=pltpu.CompilerParams(
            collective_id=0,
            dimension_semantics=("arbitrary",),
        ),
    )(Q, K, V)
    return out


# device time: 99192 ns/iter; 1.0603x vs baseline; 1.0603x over previous
import jax
import jax.numpy as jnp
from jax import lax
from jax.experimental import pallas as pl
from jax.experimental.pallas import tpu as pltpu

NPOS = 8


def kernel(Q, K, V):
    b, s, h, d = Q.shape
    scale = d ** -0.5
    nhops = NPOS - 1

    def body(q_hbm, k_hbm, v_hbm, o_hbm,
             qs, ks, vs, ostage, qT, kT, vT, krem, vrem, uring, lring,
             in_sems, o_sems, p1s,
             cwKs, cwKr, cwVs, cwVr, ccwKs, ccwKr, ccwVs, ccwVr):
        step = pl.program_id(0)
        my_x = lax.axis_index("x")
        my_y = lax.axis_index("y")
        my_z = lax.axis_index("z")
        xpartner = (1 - my_x, my_y, my_z)
        p = jnp.where(my_y == 0, my_z, 7 - my_z)

        def ring_coords(t):
            t = lax.rem(t + NPOS, NPOS)
            return (my_x, jnp.where(t < 4, 0, 1), jnp.where(t < 4, t, 7 - t))

        cw = ring_coords(p + 1)
        ccw = ring_coords(p - 1)

        def qe(k_):
            return lax.rem(p - k_ + 2 * NPOS, NPOS)

        def qo(k_):
            return lax.rem(p + k_, NPOS)

        MESH = pl.DeviceIdType.MESH

        def in_dma(j, slot):
            return [
                pltpu.make_async_copy(
                    ref.at[0, :, j, :], stage.at[slot], in_sems.at[slot, t])
                for t, (ref, stage) in enumerate(
                    [(q_hbm, qs), (k_hbm, ks), (v_hbm, vs)])
            ]

        def fetch_heads(heads):
            for j in heads:
                for c in in_dma(j, lax.rem(j, 2)):
                    c.start()

        def hop_heads(k_):
            return (2 * qe(k_), 2 * qo(k_) + 1)

        def stage_head(j):
            slot = lax.rem(j, 2)
            for c in in_dma(j, slot):
                c.wait()
            qT[j] = (qs[slot] * scale).astype(jnp.bfloat16)
            kT[j] = ks[slot].astype(jnp.bfloat16)
            vT[j] = vs[slot].astype(jnp.bfloat16)

        def local_half(j):
            slot = lax.rem(j, 4)
            s1 = lax.dot_general(
                qT[j], kT[j], (((1,), (1,)), ((), ())),
                preferred_element_type=jnp.float32,
            )
            p1 = jnp.exp(s1).astype(jnp.bfloat16)
            ones = jnp.ones((s, 128), jnp.bfloat16)
            uring[slot] = lax.dot_general(
                p1, vT[j], (((1,), (0,)), ((), ())),
                preferred_element_type=jnp.float32,
            )
            lring[slot] = lax.dot_general(
                p1, ones, (((1,), (0,)), ((), ())),
                preferred_element_type=jnp.float32,
            ).astype(jnp.bfloat16)

        def o_dma(j):
            return pltpu.make_async_copy(
                ostage.at[lax.rem(j, 4)], o_hbm.at[0, :, j, :],
                o_sems.at[lax.rem(j, 4)])

        def merge(j, reclaim_cond):
            slot = lax.rem(j, 4)

            @pl.when(reclaim_cond)
            def _():
                o_dma(j).wait()

            s2 = lax.dot_general(
                qT[j], krem[j], (((1,), (1,)), ((), ())),
                preferred_element_type=jnp.float32,
            )
            p2 = jnp.exp(s2).astype(jnp.bfloat16)
            ones = jnp.ones((s, 128), jnp.bfloat16)
            u2 = lax.dot_general(
                p2, vrem[j], (((1,), (0,)), ((), ())),
                preferred_element_type=jnp.float32,
            )
            l2 = lax.dot_general(
                p2, ones, (((1,), (0,)), ((), ())),
                preferred_element_type=jnp.float32,
            )
            ostage[slot] = (uring[slot] + u2) / (
                lring[slot][:, 0:1].astype(jnp.float32) + l2[:, 0:1])
            o_dma(j).start()

        def hop_rdmas(k_):
            he, ho = hop_heads(k_)
            kr = k_ + 1
            return [
                pltpu.make_async_remote_copy(
                    src_ref=krem.at[he], dst_ref=krem.at[he],
                    send_sem=cwKs.at[k_], recv_sem=cwKr.at[kr],
                    device_id=cw, device_id_type=MESH),
                pltpu.make_async_remote_copy(
                    src_ref=vrem.at[he], dst_ref=vrem.at[he],
                    send_sem=cwVs.at[k_], recv_sem=cwVr.at[kr],
                    device_id=cw, device_id_type=MESH),
                pltpu.make_async_remote_copy(
                    src_ref=krem.at[ho], dst_ref=krem.at[ho],
                    send_sem=ccwKs.at[k_], recv_sem=ccwKr.at[kr],
                    device_id=ccw, device_id_type=MESH),
                pltpu.make_async_remote_copy(
                    src_ref=vrem.at[ho], dst_ref=vrem.at[ho],
                    send_sem=ccwVs.at[k_], recv_sem=ccwVr.at[kr],
                    device_id=ccw, device_id_type=MESH),
            ]

        def recv_descs(k_):
            he, ho = hop_heads(k_)
            return [
                pltpu.make_async_remote_copy(
                    src_ref=krem.at[he], dst_ref=krem.at[he],
                    send_sem=cwKs.at[k_], recv_sem=cwKr.at[k_],
                    device_id=cw, device_id_type=MESH),
                pltpu.make_async_remote_copy(
                    src_ref=vrem.at[he], dst_ref=vrem.at[he],
                    send_sem=cwVs.at[k_], recv_sem=cwVr.at[k_],
                    device_id=cw, device_id_type=MESH),
                pltpu.make_async_remote_copy(
                    src_ref=krem.at[ho], dst_ref=krem.at[ho],
                    send_sem=ccwKs.at[k_], recv_sem=ccwKr.at[k_],
                    device_id=cw, device_id_type=MESH),
                pltpu.make_async_remote_copy(
                    src_ref=vrem.at[ho], dst_ref=vrem.at[ho],
                    send_sem=ccwVs.at[k_], recv_sem=ccwVr.at[k_],
                    device_id=cw, device_id_type=MESH),
            ]

        def p1_descs():
            he, ho = 2 * p, 2 * p + 1
            return [
                pltpu.make_async_remote_copy(
                    src_ref=kT.at[he], dst_ref=krem.at[he],
                    send_sem=p1s.at[0], recv_sem=cwKr.at[0],
                    device_id=xpartner, device_id_type=MESH),
                pltpu.make_async_remote_copy(
                    src_ref=vT.at[he], dst_ref=vrem.at[he],
                    send_sem=p1s.at[1], recv_sem=cwVr.at[0],
                    device_id=xpartner, device_id_type=MESH),
                pltpu.make_async_remote_copy(
                    src_ref=kT.at[ho], dst_ref=krem.at[ho],
                    send_sem=p1s.at[2], recv_sem=ccwKr.at[0],
                    device_id=xpartner, device_id_type=MESH),
                pltpu.make_async_remote_copy(
                    src_ref=vT.at[ho], dst_ref=vrem.at[ho],
                    send_sem=p1s.at[3], recv_sem=ccwVr.at[0],
                    device_id=xpartner, device_id_type=MESH),
            ]

        @pl.when(step == 0)
        def _():
            barrier_sem = pltpu.get_barrier_semaphore()
            for nbr in (xpartner, cw, ccw):
                pl.semaphore_signal(
                    barrier_sem, inc=1, device_id=nbr, device_id_type=MESH)
            pl.semaphore_wait(barrier_sem, 3)

            fetch_heads((2 * p, 2 * p + 1))
            stage_head(2 * p)
            stage_head(2 * p + 1)
            for r_ in p1_descs():
                r_.start()
            local_half(2 * p)
            local_half(2 * p + 1)
            fetch_heads(hop_heads(1))
            for j in hop_heads(1):
                stage_head(j)
                local_half(j)

        @pl.when(step < NPOS)
        def _():
            k_ = step
            @pl.when(jnp.logical_and(k_ >= 1, k_ < nhops))
            def _():
                fetch_heads(hop_heads(k_ + 1))
                for j in hop_heads(k_ + 1):
                    stage_head(j)
                    local_half(j)

            for r_ in recv_descs(k_):
                r_.wait_recv()

            @pl.when(k_ < nhops)
            def _():
                for r_ in hop_rdmas(k_):
                    r_.start()

            for j in hop_heads(k_):
                merge(j, k_ >= 2)

        @pl.when(step == 1)
        def _():
            for r_ in p1_descs():
                r_.wait_send()

        @pl.when(jnp.logical_and(2 <= step, step < NPOS))
        def _():
            for r_ in hop_rdmas(step - 2):
                r_.wait_send()

        @pl.when(step == NPOS)
        def _():
            for r_ in hop_rdmas(nhops - 1):
                r_.wait_send()
            for j4 in range(4):
                pltpu.make_async_copy(
                    ostage.at[j4], o_hbm.at[0, :, j4, :], o_sems.at[j4]
                ).wait()

    out = pl.pallas_call(
        body,
        grid=(NPOS + 1,),
        out_shape=jax.ShapeDtypeStruct((b, s, h, d), jnp.float32),
        in_specs=[pl.BlockSpec(memory_space=pl.ANY)] * 3,
        out_specs=pl.BlockSpec(memory_space=pl.ANY),
        scratch_shapes=[
            pltpu.VMEM((2, s, d), jnp.float32),
            pltpu.VMEM((2, s, d), jnp.float32),
            pltpu.VMEM((2, s, d), jnp.float32),
            pltpu.VMEM((4, s, d), jnp.float32),
            pltpu.VMEM((h, s, d), jnp.bfloat16),
            pltpu.VMEM((h, s, d), jnp.bfloat16),
            pltpu.VMEM((h, s, d), jnp.bfloat16),
            pltpu.VMEM((h, s, d), jnp.bfloat16),
            pltpu.VMEM((h, s, d), jnp.bfloat16),
            pltpu.VMEM((4, s, d), jnp.float32),
            pltpu.VMEM((4, s, 128), jnp.bfloat16),
            pltpu.SemaphoreType.DMA((2, 3)),
            pltpu.SemaphoreType.DMA((4,)),
            pltpu.SemaphoreType.DMA((4,)),
            pltpu.SemaphoreType.DMA((NPOS,)),
            pltpu.SemaphoreType.DMA((NPOS,)),
            pltpu.SemaphoreType.DMA((NPOS,)),
            pltpu.SemaphoreType.DMA((NPOS,)),
            pltpu.SemaphoreType.DMA((NPOS,)),
            pltpu.SemaphoreType.DMA((NPOS,)),
            pltpu.SemaphoreType.DMA((NPOS,)),
            pltpu.SemaphoreType.DMA((NPOS,)),
        ],
        compiler_params=pltpu.CompilerParams(
            collective_id=0,
            dimension_semantics=("arbitrary",),
        ),
    )(Q, K, V)
    return out
